# Initial kernel scaffold; baseline (speedup 1.0000x reference)
import jax
import jax.numpy as jnp
from jax import lax
from jax.experimental import pallas as pl
from jax.experimental.pallas import tpu as pltpu
from jax.experimental.pallas import tpu_sc as plsc

NC = 2
NS = 16
NW = NC * NS
L = 16

B = 16384
J = 20
EMB = 64

B_W = B // NW
CB = 32
NCH = B_W // CB
NEG_ROWS = CB * J
G = 128
NG = NEG_ROWS // G


def _sgns_body(c_h, o_h, n_h, vemb, uemb, pos_h, negout_h,
               cidx, oidx, nidx, vrows, orows, nrows, posb, negb, sem):
    w = lax.axis_index("s") * NC + lax.axis_index("c")

    pltpu.sync_copy(c_h.at[w], cidx)
    pltpu.sync_copy(o_h.at[w], oidx)
    pltpu.sync_copy(n_h.at[w], nidx)

    lane = lax.iota(jnp.int32, (L,))
    last = lane == (L - 1)

    def chunk(ch, carry):
        cps = [
            pltpu.async_copy(vemb.at[cidx.at[ch]], vrows, sem),
            pltpu.async_copy(uemb.at[oidx.at[ch]], orows, sem),
        ]
        for k in range(NG):
            cps.append(pltpu.async_copy(uemb.at[nidx.at[ch * NG + k]],
                                        nrows.at[pl.ds(k * G, G)], sem))
        for cp in cps:
            cp.wait()

        def bbody(bl, c2):
            v0 = vrows[bl, pl.ds(0, L)]
            v1 = vrows[bl, pl.ds(L, L)]
            v2 = vrows[bl, pl.ds(2 * L, L)]
            v3 = vrows[bl, pl.ds(3 * L, L)]
            b_abs = ch * CB + bl

            def dot_store(rref, row, out_ref, pos):
                acc = rref[row, pl.ds(0, L)] * v0
                acc = acc + rref[row, pl.ds(L, L)] * v1
                acc = acc + rref[row, pl.ds(2 * L, L)] * v2
                acc = acc + rref[row, pl.ds(3 * L, L)] * v3
                s = plsc.cumsum(acc)
                idx = jnp.full((L,), pos, dtype=jnp.int32)
                plsc.store_scatter(out_ref, [idx], s, mask=last)

            dot_store(orows, bl, posb, b_abs)
            for j in range(J):
                dot_store(nrows, bl * J + j, negb, b_abs * J + j)
            return c2

        lax.fori_loop(0, CB, bbody, 0)
        return carry

    lax.fori_loop(0, NCH, chunk, 0)

    def sig_pos(i, c2):
        x = posb[pl.ds(i * L, L)]
        posb[pl.ds(i * L, L)] = 1.0 / (1.0 + jnp.exp(-x))
        return c2

    def sig_neg(i, c2):
        x = negb[pl.ds(i * L, L)]
        negb[pl.ds(i * L, L)] = 1.0 / (1.0 + jnp.exp(x))
        return c2

    lax.fori_loop(0, B_W // L, sig_pos, 0)
    lax.fori_loop(0, B_W * J // L, sig_neg, 0)

    pltpu.sync_copy(posb, pos_h.at[w])
    pltpu.sync_copy(negb, negout_h.at[w])


@jax.jit
def _sgns(c_h, o_h, n_h, vemb, uemb):
    mesh = plsc.VectorSubcoreMesh(core_axis_name="c", subcore_axis_name="s",
                                  num_cores=NC, num_subcores=NS)
    f = pl.kernel(
        _sgns_body,
        out_type=(
            jax.ShapeDtypeStruct((NW, B_W), jnp.float32),
            jax.ShapeDtypeStruct((NW, B_W * J), jnp.float32),
        ),
        mesh=mesh,
        scratch_types=[
            pltpu.VMEM((NCH, CB), jnp.int32),
            pltpu.VMEM((NCH, CB), jnp.int32),
            pltpu.VMEM((NCH * NG, G), jnp.int32),
            pltpu.VMEM((CB, EMB), jnp.float32),
            pltpu.VMEM((CB, EMB), jnp.float32),
            pltpu.VMEM((NEG_ROWS, EMB), jnp.float32),
            pltpu.VMEM((B_W,), jnp.float32),
            pltpu.VMEM((B_W * J,), jnp.float32),
            pltpu.SemaphoreType.DMA,
        ],
    )
    return f(c_h, o_h, n_h, vemb, uemb)


def kernel(c, o, neg, vEmbedding, uEmbedding):
    c_h = c.reshape(NW, NCH, CB).astype(jnp.int32)
    o_h = o.reshape(NW, NCH, CB).astype(jnp.int32)
    n_h = neg.reshape(NW, NCH * NG, G).astype(jnp.int32)
    pos, negout = _sgns(c_h, o_h, n_h, vEmbedding, uEmbedding)
    return pos.reshape(B, 1), negout.reshape(B, J, 1)


# speedup vs baseline: 4.7436x; 4.7436x over previous
import jax
import jax.numpy as jnp
from jax import lax
from jax.experimental import pallas as pl
from jax.experimental.pallas import tpu as pltpu
from jax.experimental.pallas import tpu_sc as plsc

NC = 2
NS = 16
NW = NC * NS
L = 16

B = 16384
J = 20
EMB = 64

B_W = B // NW
CB = 32
NCH = B_W // CB
NEG_ROWS = CB * J
G = 128
NG = NEG_ROWS // G


def _sgns_body(c_h, o_h, n_h, vemb, uemb, pos_h, negout_h,
               cidx, oidx, nidx, vrows, orows, nrows, posb, negb, sem):
    w = lax.axis_index("s") * NC + lax.axis_index("c")

    pltpu.sync_copy(c_h.at[w], cidx)
    pltpu.sync_copy(o_h.at[w], oidx)
    pltpu.sync_copy(n_h.at[w], nidx)

    lane = lax.iota(jnp.int32, L)
    last = lane == (L - 1)

    def chunk(ch, carry):
        cps = [
            pltpu.async_copy(vemb.at[cidx.at[ch]], vrows, sem),
            pltpu.async_copy(uemb.at[oidx.at[ch]], orows, sem),
        ]
        for k in range(NG):
            cps.append(pltpu.async_copy(uemb.at[nidx.at[ch * NG + k]],
                                        nrows.at[pl.ds(k * G, G)], sem))
        for cp in cps:
            cp.wait()

        def bbody(bl, c2):
            v0 = vrows[bl, pl.ds(0, L)]
            v1 = vrows[bl, pl.ds(L, L)]
            v2 = vrows[bl, pl.ds(2 * L, L)]
            v3 = vrows[bl, pl.ds(3 * L, L)]
            b_abs = ch * CB + bl

            def dot_store(rref, row, out_ref, pos):
                acc = rref[row, pl.ds(0, L)] * v0
                acc = acc + rref[row, pl.ds(L, L)] * v1
                acc = acc + rref[row, pl.ds(2 * L, L)] * v2
                acc = acc + rref[row, pl.ds(3 * L, L)] * v3
                s = plsc.cumsum(acc)
                idx = jnp.full((L,), pos, dtype=jnp.int32)
                plsc.store_scatter(out_ref, [idx], s, mask=last)

            dot_store(orows, bl, posb, b_abs)
            for j in range(J):
                dot_store(nrows, bl * J + j, negb, b_abs * J + j)
            return c2

        lax.fori_loop(0, CB, bbody, 0)
        return carry

    lax.fori_loop(0, NCH, chunk, 0)

    def sig_pos(i, c2):
        x = posb[pl.ds(i * L, L)]
        posb[pl.ds(i * L, L)] = 1.0 / (1.0 + jnp.exp(-x))
        return c2

    def sig_neg(i, c2):
        x = negb[pl.ds(i * L, L)]
        negb[pl.ds(i * L, L)] = 1.0 / (1.0 + jnp.exp(x))
        return c2

    lax.fori_loop(0, B_W // L, sig_pos, 0)
    lax.fori_loop(0, B_W * J // L, sig_neg, 0)

    pltpu.sync_copy(posb, pos_h.at[w])
    pltpu.sync_copy(negb, negout_h.at[w])


@jax.jit
def _sgns(c_h, o_h, n_h, vemb, uemb):
    mesh = plsc.VectorSubcoreMesh(core_axis_name="c", subcore_axis_name="s",
                                  num_cores=NC, num_subcores=NS)
    f = pl.kernel(
        _sgns_body,
        out_type=(
            jax.ShapeDtypeStruct((NW, B_W), jnp.float32),
            jax.ShapeDtypeStruct((NW, B_W * J), jnp.float32),
        ),
        mesh=mesh,
        scratch_types=[
            pltpu.VMEM((NCH, CB), jnp.int32),
            pltpu.VMEM((NCH, CB), jnp.int32),
            pltpu.VMEM((NCH * NG, G), jnp.int32),
            pltpu.VMEM((CB, EMB), jnp.float32),
            pltpu.VMEM((CB, EMB), jnp.float32),
            pltpu.VMEM((NEG_ROWS, EMB), jnp.float32),
            pltpu.VMEM((B_W,), jnp.float32),
            pltpu.VMEM((B_W * J,), jnp.float32),
            pltpu.SemaphoreType.DMA,
        ],
        compiler_params=pltpu.CompilerParams(needs_layout_passes=False,
                                             use_tc_tiling_on_sc=False),
    )
    return f(c_h, o_h, n_h, vemb, uemb)


def kernel(c, o, neg, vEmbedding, uEmbedding):
    c_h = c.reshape(NW, NCH, CB).astype(jnp.int32)
    o_h = o.reshape(NW, NCH, CB).astype(jnp.int32)
    n_h = neg.reshape(NW, NCH * NG, G).astype(jnp.int32)
    pos, negout = _sgns(c_h, o_h, n_h, vEmbedding, uEmbedding)
    return pos.reshape(B, 1), negout.reshape(B, J, 1)
